# baseline (speedup 1.0000x reference)
import jax
import jax.numpy as jnp
from jax import lax
from jax.experimental import pallas as pl
from jax.experimental.pallas import tpu as pltpu

S = 4096
E = 16
CAP = 512
_BLK = 256


def _cumsum_tokens(x):
    k = 1
    while k < S:
        shifted = jnp.concatenate(
            [jnp.zeros((k, E), dtype=x.dtype), x[: S - k, :]], axis=0
        )
        x = x + shifted
        k *= 2
    return x


def _routing(logits, noise):
    neg_min = jnp.finfo(jnp.float32).min
    eps = jnp.finfo(jnp.float32).eps

    rowmax = jnp.max(logits, axis=1, keepdims=True)
    ex = jnp.exp(logits - rowmax)
    sumex = jnp.sum(ex, axis=1, keepdims=True)
    gates = ex / sumex

    iota_e = lax.broadcasted_iota(jnp.int32, (S, E), 1)
    e1 = jnp.min(jnp.where(logits == rowmax, iota_e, E), axis=1, keepdims=True)
    mask1 = iota_e == e1
    g1 = jnp.max(gates, axis=1, keepdims=True)

    lw = logits + noise
    masked = jnp.where(mask1, neg_min, lw)
    m2 = jnp.max(masked, axis=1, keepdims=True)
    e2 = jnp.min(jnp.where(masked == m2, iota_e, E), axis=1, keepdims=True)
    mask2 = iota_e == e2
    g2 = jnp.sum(jnp.where(mask2, gates, 0.0), axis=1, keepdims=True)

    m1f = mask1.astype(jnp.float32)
    m2f = mask2.astype(jnp.float32)
    c1 = _cumsum_tokens(m1f)
    c2 = _cumsum_tokens(m2f)
    counts1 = jnp.sum(m1f, axis=0, keepdims=True)
    loc1 = jnp.sum((c1 - 1.0) * m1f, axis=1, keepdims=True)
    loc2 = jnp.sum((c2 - 1.0 + counts1) * m2f, axis=1, keepdims=True)

    g1k = jnp.where(loc1 < float(CAP), g1, 0.0)
    g2k = jnp.where(loc2 < float(CAP), g2, 0.0)
    denom = jnp.maximum(g1k + g2k, eps)
    w1 = g1k / denom
    w2 = g2k / denom

    scal = jnp.concatenate(
        [w1, w2, e1.astype(jnp.float32), e2.astype(jnp.float32), loc1, loc2,
         w1, w2],
        axis=1,
    )
    gplanes = jnp.concatenate(
        [jnp.where(mask1, w1, 0.0), jnp.where(mask2, w2, 0.0)], axis=1
    )

    me = jnp.sum(gates, axis=0, keepdims=True) * (1.0 / S)
    ce = counts1 * (1.0 / S)
    laux = jnp.sum(me * ce, axis=1, keepdims=True) * float(E)
    return scal, gplanes, laux


def _routing_kernel(logits_ref, noise_ref, scal_ref, gpl_ref, laux_ref):
    scal, gplanes, laux = _routing(logits_ref[...], noise_ref[...])
    scal_ref[...] = scal
    gpl_ref[...] = gplanes
    laux_ref[...] = laux


def _expand_kernel(scal_ref, gpl_ref, comb_ref, mask_ref):
    sc = scal_ref[...]
    gp = gpl_ref[...]
    l1 = sc[:, 4:5].astype(jnp.int32)
    l2 = sc[:, 5:6].astype(jnp.int32)
    g1 = gp[:, :E].reshape(_BLK, E, 1)
    g2 = gp[:, E:].reshape(_BLK, E, 1)
    iota_c = lax.broadcasted_iota(jnp.int32, (_BLK, CAP), 1)
    oh1 = (iota_c == l1).astype(jnp.float32).reshape(_BLK, 1, CAP)
    oh2 = (iota_c == l2).astype(jnp.float32).reshape(_BLK, 1, CAP)
    comb = g1 * oh1 + g2 * oh2
    comb_ref[...] = comb
    mask_ref[...] = (comb != 0.0).astype(jnp.int8)


def kernel(logits, noise):
    scal, gpl, laux = pl.pallas_call(
        _routing_kernel,
        out_shape=(
            jax.ShapeDtypeStruct((S, 8), jnp.float32),
            jax.ShapeDtypeStruct((S, 2 * E), jnp.float32),
            jax.ShapeDtypeStruct((1, 1), jnp.float32),
        ),
    )(logits, noise)

    comb, mask = pl.pallas_call(
        _expand_kernel,
        grid=(S // _BLK,),
        in_specs=[
            pl.BlockSpec((_BLK, 8), lambda i: (i, 0)),
            pl.BlockSpec((_BLK, 2 * E), lambda i: (i, 0)),
        ],
        out_specs=(
            pl.BlockSpec((_BLK, E, CAP), lambda i: (i, 0, 0)),
            pl.BlockSpec((_BLK, E, CAP), lambda i: (i, 0, 0)),
        ),
        out_shape=(
            jax.ShapeDtypeStruct((S, E, CAP), jnp.float32),
            jax.ShapeDtypeStruct((S, E, CAP), jnp.int8),
        ),
    )(scal, gpl)

    return (laux[0, 0], comb, mask.astype(jnp.bool_))


# speedup vs baseline: 1.0237x; 1.0237x over previous
import jax
import jax.numpy as jnp
from jax import lax
from jax.experimental import pallas as pl
from jax.experimental.pallas import tpu as pltpu

S = 4096
E = 16
CAP = 512
_BLK = 256


def _cumsum_tokens(x):
    k = 1
    while k < S:
        shifted = jnp.concatenate(
            [jnp.zeros((k, E), dtype=x.dtype), x[: S - k, :]], axis=0
        )
        x = x + shifted
        k *= 2
    return x


def _routing(logits, noise):
    neg_min = jnp.finfo(jnp.float32).min
    eps = jnp.finfo(jnp.float32).eps

    rowmax = jnp.max(logits, axis=1, keepdims=True)
    ex = jnp.exp(logits - rowmax)
    sumex = jnp.sum(ex, axis=1, keepdims=True)
    gates = ex / sumex

    iota_e = lax.broadcasted_iota(jnp.int32, (S, E), 1)
    e1 = jnp.min(jnp.where(logits == rowmax, iota_e, E), axis=1, keepdims=True)
    mask1 = iota_e == e1
    g1 = jnp.max(gates, axis=1, keepdims=True)

    lw = logits + noise
    masked = jnp.where(mask1, neg_min, lw)
    m2 = jnp.max(masked, axis=1, keepdims=True)
    e2 = jnp.min(jnp.where(masked == m2, iota_e, E), axis=1, keepdims=True)
    mask2 = iota_e == e2
    g2 = jnp.sum(jnp.where(mask2, gates, 0.0), axis=1, keepdims=True)

    m1f = mask1.astype(jnp.float32)
    m2f = mask2.astype(jnp.float32)
    c1 = _cumsum_tokens(m1f)
    c2 = _cumsum_tokens(m2f)
    counts1 = jnp.sum(m1f, axis=0, keepdims=True)
    loc1 = jnp.sum((c1 - 1.0) * m1f, axis=1, keepdims=True)
    loc2 = jnp.sum((c2 - 1.0 + counts1) * m2f, axis=1, keepdims=True)

    g1k = jnp.where(loc1 < float(CAP), g1, 0.0)
    g2k = jnp.where(loc2 < float(CAP), g2, 0.0)
    denom = jnp.maximum(g1k + g2k, eps)
    w1 = g1k / denom
    w2 = g2k / denom

    scal = jnp.concatenate(
        [w1, w2, e1.astype(jnp.float32), e2.astype(jnp.float32), loc1, loc2,
         w1, w2],
        axis=1,
    )
    gplanes = jnp.concatenate(
        [jnp.where(mask1, w1, 0.0), jnp.where(mask2, w2, 0.0)], axis=1
    )

    me = jnp.sum(gates, axis=0, keepdims=True) * (1.0 / S)
    ce = counts1 * (1.0 / S)
    laux = jnp.sum(me * ce, axis=1, keepdims=True) * float(E)
    return scal, gplanes, laux


def _fused_kernel(logits_ref, noise_ref, comb_ref, mask_ref, laux_ref,
                  scal_ref, gpl_ref):
    i = pl.program_id(0)

    @pl.when(i == 0)
    def _():
        scal, gplanes, laux = _routing(logits_ref[...], noise_ref[...])
        scal_ref[...] = scal
        gpl_ref[...] = gplanes
        laux_ref[...] = laux

    sc = scal_ref[pl.ds(i * _BLK, _BLK), :]
    gp = gpl_ref[pl.ds(i * _BLK, _BLK), :]
    l1 = sc[:, 4:5].astype(jnp.int32)
    l2 = sc[:, 5:6].astype(jnp.int32)
    g1 = gp[:, :E].reshape(_BLK, E, 1)
    g2 = gp[:, E:].reshape(_BLK, E, 1)
    iota_c = lax.broadcasted_iota(jnp.int32, (_BLK, CAP), 1)
    oh1 = (iota_c == l1).astype(jnp.float32).reshape(_BLK, 1, CAP)
    oh2 = (iota_c == l2).astype(jnp.float32).reshape(_BLK, 1, CAP)
    comb = g1 * oh1 + g2 * oh2
    comb_ref[...] = comb
    mask_ref[...] = (comb != 0.0).astype(jnp.int8)


def kernel(logits, noise):
    comb, mask, laux = pl.pallas_call(
        _fused_kernel,
        grid=(S // _BLK,),
        in_specs=[
            pl.BlockSpec((S, E), lambda i: (0, 0)),
            pl.BlockSpec((S, E), lambda i: (0, 0)),
        ],
        out_specs=(
            pl.BlockSpec((_BLK, E, CAP), lambda i: (i, 0, 0)),
            pl.BlockSpec((_BLK, E, CAP), lambda i: (i, 0, 0)),
            pl.BlockSpec((1, 1), lambda i: (0, 0)),
        ),
        out_shape=(
            jax.ShapeDtypeStruct((S, E, CAP), jnp.float32),
            jax.ShapeDtypeStruct((S, E, CAP), jnp.int8),
            jax.ShapeDtypeStruct((1, 1), jnp.float32),
        ),
        scratch_shapes=[
            pltpu.VMEM((S, 8), jnp.float32),
            pltpu.VMEM((S, 2 * E), jnp.float32),
        ],
    )(logits, noise)

    return (laux[0, 0], comb, mask.astype(jnp.bool_))
